# baseline (speedup 1.0000x reference)
import jax
import jax.numpy as jnp
from jax.experimental import pallas as pl

EPS_ = -0.03
BLOCK = 8192


def _vel_block(x_ref, w1_ref, b1_ref, w2_ref, b2_ref, out_ref):
    x = x_ref[...]
    h = jnp.dot(x, w1_ref[...], preferred_element_type=jnp.float32)
    h = jnp.maximum(h + b1_ref[...], 0.0)
    v = jnp.dot(h, w2_ref[...], preferred_element_type=jnp.float32)
    v = v + b2_ref[...]
    pts = x[:, :3]
    mask = jnp.any((pts < -1.0 + EPS_) | (pts > 1.0 - EPS_), axis=1,
                   keepdims=True)
    out_ref[...] = jnp.where(mask, 0.0, v)


def kernel(xt, W1, b1, W2, b2):
    n = xt.shape[0]
    grid = (n // BLOCK,)
    return pl.pallas_call(
        _vel_block,
        grid=grid,
        in_specs=[
            pl.BlockSpec((BLOCK, 4), lambda i: (i, 0)),
            pl.BlockSpec((4, 64), lambda i: (0, 0)),
            pl.BlockSpec((1, 64), lambda i: (0, 0)),
            pl.BlockSpec((64, 3), lambda i: (0, 0)),
            pl.BlockSpec((1, 3), lambda i: (0, 0)),
        ],
        out_specs=pl.BlockSpec((BLOCK, 3), lambda i: (i, 0)),
        out_shape=jax.ShapeDtypeStruct((n, 3), xt.dtype),
    )(xt, W1, b1.reshape(1, 64), W2, b2.reshape(1, 3))
